# baseline (speedup 1.0000x reference)
import jax
import jax.numpy as jnp
from jax.experimental import pallas as pl

B = 128
NPG = 39
N = B * NPG
E = 49920
ET = E + N


def _mm_body(x_ref, w_ref, o_ref):
    o_ref[...] = jnp.dot(x_ref[...], w_ref[...], preferred_element_type=jnp.float32)


def _mm(x, w):
    return pl.pallas_call(
        _mm_body,
        out_shape=jax.ShapeDtypeStruct((x.shape[0], w.shape[1]), jnp.float32),
    )(x, w)


def _conv2d(x, w, b, pad):
    y = jax.lax.conv_general_dilated(
        x, w, (1, 1), [(pad, pad), (pad, pad)],
        dimension_numbers=('NCHW', 'OIHW', 'NCHW'))
    return y + b[None, :, None, None]


def _bn(x, g, bt):
    return x * (g / jnp.sqrt(1.0 + 1e-5))[None, :, None, None] + bt[None, :, None, None]


def _gru(x, wi, wh, bi, bh):
    def step(h, xt):
        gi = xt @ wi.T + bi
        gh = h @ wh.T + bh
        r = jax.nn.sigmoid(gi[:, 0:1] + gh[:, 0:1])
        z = jax.nn.sigmoid(gi[:, 1:2] + gh[:, 1:2])
        n = jnp.tanh(gi[:, 2:3] + r * gh[:, 2:3])
        h2 = (1.0 - z) * n + z * h
        return h2, h2
    h0 = jnp.zeros((x.shape[0], 1), x.dtype)
    _, ys = jax.lax.scan(step, h0, jnp.swapaxes(x, 0, 1))
    return jnp.swapaxes(ys, 0, 1)[..., 0]


def _gatv2(x, src, dst, wl, wr, att, bias, heads, oc):
    loop = jnp.arange(N)
    s = jnp.concatenate([src, loop])
    d = jnp.concatenate([dst, loop])
    xl = _mm(x, wl).reshape(N, heads, oc)
    xr = _mm(x, wr).reshape(N, heads, oc)
    e = jax.nn.leaky_relu(xl[s] + xr[d], 0.2)
    logit = (e * att[None, :, :]).sum(-1)
    m = jax.lax.stop_gradient(jax.ops.segment_max(logit, d, num_segments=N))
    ex = jnp.exp(logit - m[d])
    den = jax.ops.segment_sum(ex, d, num_segments=N)
    alpha = ex / jnp.maximum(den, 1e-16)[d]
    out = jax.ops.segment_sum(alpha[..., None] * xl[s], d, num_segments=N)
    return out.reshape(N, heads * oc) + bias


def kernel(edge_index_batch, ve_matrix_batch, ac_matrix_batch, man_matrix_batch, mask_view_batch, graph_matrix, conv1_w, conv1_b, bn1_g, bn1_b, conv2_w, conv2_b, bn2_g, bn2_b, gru_wi, gru_wh, gru_bi, gru_bh, gat1_wl, gat1_wr, gat1_att, gat1_bias, gat2_wl, gat2_wr, gat2_att, gat2_bias):
    man = jnp.where(jnp.isnan(man_matrix_batch), 0.0, man_matrix_batch)
    ac = jnp.where(jnp.isnan(ac_matrix_batch), 0.0, ac_matrix_batch)
    ve = jnp.where(jnp.isnan(ve_matrix_batch), 0.0, ve_matrix_batch)
    cm = jnp.stack([man, ac, ve], axis=1)
    cm = jax.nn.relu(_bn(_conv2d(cm, conv1_w, conv1_b, 0), bn1_g, bn1_b))
    cm = _bn(_conv2d(cm, conv2_w, conv2_b, 1), bn2_g, bn2_b)
    b, c, hdim, wdim = cm.shape
    seq = jnp.transpose(cm, (0, 3, 2, 1)).reshape(b * wdim, hdim, c)
    g = _gru(seq, gru_wi, gru_wh, gru_bi, gru_bh)
    conv_enc1 = jnp.transpose(g.reshape(b, wdim, hdim), (0, 2, 1))
    mk = mask_view_batch.reshape(b, -1)[:, None, :]
    conv_enc2 = conv_enc1 * mk
    man2 = man * mk
    gm = jnp.concatenate([man2, conv_enc2], axis=1)
    x = jnp.transpose(gm, (0, 2, 1)).reshape(-1, 78)
    ei = edge_index_batch.reshape(2, -1)
    h = _gatv2(x, ei[0], ei[1], gat1_wl, gat1_wr, gat1_att, gat1_bias, 8, 64)
    h = jax.nn.elu(h)
    h = _gatv2(h, ei[0], ei[1], gat2_wl, gat2_wr, gat2_att, gat2_bias, 1, 64)
    return h.reshape(128, 39, 64)


# speedup vs baseline: 1.0687x; 1.0687x over previous
import jax
import jax.numpy as jnp
from jax.experimental import pallas as pl

B = 128
NPG = 39
N = B * NPG
E = 49920
ET = E + N


def _frontend_body(manT, acT, veT, mkT, c1w, c1b, b1g, b1b, c2w, c2b, b2g, b2b,
                   gwi, gwh, gbi, gbh, out_ref):
    s1 = b1g[...] / jnp.sqrt(1.0 + 1e-5)
    s2 = b2g[...] / jnp.sqrt(1.0 + 1e-5)
    man = manT[...]
    man = jnp.where(man != man, 0.0, man)
    ac = acT[...]
    ac = jnp.where(ac != ac, 0.0, ac)
    ve = veT[...]
    ve = jnp.where(ve != ve, 0.0, ve)
    mk = mkT[...]
    M = [man, ac, ve]
    P = []
    for o in range(8):
        acc = jnp.zeros((39, 39, 128), jnp.float32)
        for c in range(3):
            acc = acc + (s1[o] * c1w[o, c, 0, 0]) * M[c]
        acc = acc + (s1[o] * c1b[o] + b1b[o])
        acc = jnp.maximum(acc, 0.0)
        P.append(jnp.pad(acc, ((1, 1), (1, 1), (0, 0))))
    S2 = []
    for o in range(16):
        acc = jnp.zeros((39, 39, 128), jnp.float32)
        for c in range(8):
            for di in range(3):
                for dj in range(3):
                    acc = acc + (s2[o] * c2w[o, c, di, dj]) * jax.lax.slice(
                        P[c], (di, dj, 0), (di + 39, dj + 39, 128))
        acc = acc + (s2[o] * c2b[o] + b2b[o])
        S2.append(acc)
    for f in range(39):
        out_ref[f] = M[0][f] * mk
    h = jnp.zeros((39, 128), jnp.float32)
    for t in range(39):
        g0 = jnp.full((39, 128), gbi[0])
        g1 = jnp.full((39, 128), gbi[1])
        g2 = jnp.full((39, 128), gbi[2])
        for c in range(16):
            xt = S2[c][t]
            g0 = g0 + gwi[0, c] * xt
            g1 = g1 + gwi[1, c] * xt
            g2 = g2 + gwi[2, c] * xt
        gh0 = gwh[0, 0] * h + gbh[0]
        gh1 = gwh[1, 0] * h + gbh[1]
        gh2 = gwh[2, 0] * h + gbh[2]
        r = jax.nn.sigmoid(g0 + gh0)
        z = jax.nn.sigmoid(g1 + gh1)
        nn_ = jnp.tanh(g2 + r * gh2)
        h = (1.0 - z) * nn_ + z * h
        out_ref[39 + t] = h * mk


def _frontend(man, ac, ve, mask, c1w, c1b, b1g, b1b, c2w, c2b, b2g, b2b,
              gwi, gwh, gbi, gbh):
    manT = jnp.transpose(man, (1, 2, 0))
    acT = jnp.transpose(ac, (1, 2, 0))
    veT = jnp.transpose(ve, (1, 2, 0))
    mkT = jnp.transpose(mask.reshape(B, 39), (1, 0))
    xT = pl.pallas_call(
        _frontend_body,
        out_shape=jax.ShapeDtypeStruct((78, 39, 128), jnp.float32),
    )(manT, acT, veT, mkT, c1w, c1b, b1g, b1b, c2w, c2b, b2g, b2b,
      gwi, gwh, gbi, gbh)
    return jnp.transpose(xT, (2, 1, 0)).reshape(N, 78)


def _mm_body(x_ref, w_ref, o_ref):
    o_ref[...] = jnp.dot(x_ref[...], w_ref[...], preferred_element_type=jnp.float32)


def _mm(x, w):
    return pl.pallas_call(
        _mm_body,
        out_shape=jax.ShapeDtypeStruct((x.shape[0], w.shape[1]), jnp.float32),
    )(x, w)


def _gatv2(x, s, d, wl, wr, att, bias, heads, oc):
    xl = _mm(x, wl).reshape(N, heads, oc)
    xr = _mm(x, wr).reshape(N, heads, oc)
    e = jax.nn.leaky_relu(xl[s] + xr[d], 0.2)
    logit = (e * att[None, :, :]).sum(-1)
    m = jnp.max(logit, axis=0)
    ex = jnp.exp(logit - m[None, :])
    den = jax.ops.segment_sum(ex, d, num_segments=N)
    alpha = ex / jnp.maximum(den, 1e-16)[d]
    out = jax.ops.segment_sum(alpha[..., None] * xl[s], d, num_segments=N)
    return out.reshape(N, heads * oc) + bias


def kernel(edge_index_batch, ve_matrix_batch, ac_matrix_batch, man_matrix_batch, mask_view_batch, graph_matrix, conv1_w, conv1_b, bn1_g, bn1_b, conv2_w, conv2_b, bn2_g, bn2_b, gru_wi, gru_wh, gru_bi, gru_bh, gat1_wl, gat1_wr, gat1_att, gat1_bias, gat2_wl, gat2_wr, gat2_att, gat2_bias):
    x = _frontend(man_matrix_batch, ac_matrix_batch, ve_matrix_batch,
                  mask_view_batch, conv1_w, conv1_b, bn1_g, bn1_b,
                  conv2_w, conv2_b, bn2_g, bn2_b,
                  gru_wi, gru_wh, gru_bi, gru_bh)
    ei = edge_index_batch.reshape(2, -1)
    loop = jnp.arange(N, dtype=ei.dtype)
    s = jnp.concatenate([ei[0], loop])
    d = jnp.concatenate([ei[1], loop])
    h = _gatv2(x, s, d, gat1_wl, gat1_wr, gat1_att, gat1_bias, 8, 64)
    h = jax.nn.elu(h)
    h = _gatv2(h, s, d, gat2_wl, gat2_wr, gat2_att, gat2_bias, 1, 64)
    return h.reshape(128, 39, 64)
